# baseline (speedup 1.0000x reference)
import functools

import jax
import jax.numpy as jnp
from jax import lax
from jax.experimental import pallas as pl
from jax.experimental.pallas import tpu as pltpu
from jax.experimental.pallas import tpu_sc as plsc

_NC = 2
_NS = 16
_NW = _NC * _NS

_B = 32768
_D = 1024
_BPW = _B // _NW
_C = 32
_G = _BPW // _C
_NBUF = 3


def _gather_body(pos_hbm, enc_hbm, out_hbm, idx_v, *scratch):
    rows = scratch[:_NBUF]
    gsems = scratch[_NBUF:2 * _NBUF]
    ssems = scratch[2 * _NBUF:3 * _NBUF]

    c = lax.axis_index("c")
    s = lax.axis_index("s")
    wid = s * _NC + c
    base = pl.multiple_of(wid * _BPW, 8)

    pltpu.sync_copy(pos_hbm.at[pl.ds(base, _BPW)], idx_v)

    def start_gather(off, b):
        pltpu.async_copy(enc_hbm.at[idx_v.at[pl.ds(off, _C)]], rows[b], gsems[b])

    def wait_gather(b):
        pltpu.make_async_copy(enc_hbm.at[idx_v.at[pl.ds(0, _C)]], rows[b],
                              gsems[b]).wait()

    def start_store(off, b):
        pltpu.async_copy(rows[b], out_hbm.at[pl.ds(base + off, _C)], ssems[b])

    def drain_store(b):
        pltpu.make_async_copy(rows[b], out_hbm.at[pl.ds(0, _C)], ssems[b]).wait()

    start_gather(0, 0)
    start_gather(_C, 1)
    start_gather(2 * _C, 2)
    wait_gather(0)
    start_store(0, 0)
    wait_gather(1)
    start_store(_C, 1)

    def group(t, carry):
        for j in range(_NBUF):
            g = 3 * t + 2 + j
            b = (2 + j) % _NBUF
            off = pl.multiple_of(g * _C, _C)
            off_nxt = pl.multiple_of((g + 1) * _C, _C)
            wait_gather(b)
            start_store(off, b)
            drain_store((b + 1) % _NBUF)
            start_gather(off_nxt, (b + 1) % _NBUF)
        return carry

    lax.fori_loop(0, (_G - 5) // _NBUF, group, 0)

    for g in (_G - 3, _G - 2):
        b = g % _NBUF
        wait_gather(b)
        start_store(g * _C, b)
        drain_store((b + 1) % _NBUF)
        start_gather((g + 1) * _C, (b + 1) % _NBUF)
    b_last = (_G - 1) % _NBUF
    wait_gather(b_last)
    start_store((_G - 1) * _C, b_last)
    drain_store((b_last + 1) % _NBUF)
    drain_store((b_last + 2) % _NBUF)
    drain_store(b_last)


@functools.partial(jax.jit, static_argnames=())
def _gather(pos_flat, encoding):
    mesh = plsc.VectorSubcoreMesh(core_axis_name="c", subcore_axis_name="s")
    run = pl.kernel(
        _gather_body,
        out_type=jax.ShapeDtypeStruct((_B, _D), jnp.float32),
        mesh=mesh,
        scratch_types=(
            [pltpu.VMEM((_BPW,), jnp.int32)]
            + [pltpu.VMEM((_C, _D), jnp.float32) for _ in range(_NBUF)]
            + [pltpu.SemaphoreType.DMA for _ in range(2 * _NBUF)]
        ),
    )
    return run(pos_flat, encoding)


def kernel(pos, encoding):
    b, s = pos.shape
    out = _gather(pos.reshape(-1), encoding)
    return out.reshape(b, s, encoding.shape[1])


# speedup vs baseline: 1.0295x; 1.0295x over previous
import functools

import jax
import jax.numpy as jnp
from jax import lax
from jax.experimental import pallas as pl
from jax.experimental.pallas import tpu as pltpu
from jax.experimental.pallas import tpu_sc as plsc

_NC = 2
_NS = 16
_NW = _NC * _NS

_B = 32768
_D = 1024
_BPW = _B // _NW
_C = 16
_G = _BPW // _C
_NBUF = 6
_K = 3


def _gather_body(pos_hbm, enc_hbm, out_hbm, idx_v, *scratch):
    rows = scratch[:_NBUF]
    gsems = scratch[_NBUF:2 * _NBUF]
    ssems = scratch[2 * _NBUF:3 * _NBUF]

    c = lax.axis_index("c")
    s = lax.axis_index("s")
    wid = s * _NC + c
    base = pl.multiple_of(wid * _BPW, 8)

    pltpu.sync_copy(pos_hbm.at[pl.ds(base, _BPW)], idx_v)

    def start_gather(off, b):
        pltpu.async_copy(enc_hbm.at[idx_v.at[pl.ds(off, _C)]], rows[b], gsems[b])

    def wait_gather(b):
        pltpu.make_async_copy(enc_hbm.at[idx_v.at[pl.ds(0, _C)]], rows[b],
                              gsems[b]).wait()

    def start_store(off, b):
        pltpu.async_copy(rows[b], out_hbm.at[pl.ds(base + off, _C)], ssems[b])

    def drain_store(b):
        pltpu.make_async_copy(rows[b], out_hbm.at[pl.ds(0, _C)], ssems[b]).wait()

    for g in range(_K):
        start_gather(g * _C, g)
    for g in range(_K):
        wait_gather(g)
        start_store(g * _C, g)
        start_gather((g + _K) * _C, g + _K)

    _NGROUPS = (_G - 2 * _K - 1) // _NBUF

    def group(t, carry):
        for j in range(_NBUF):
            g = _NBUF * t + _K + j
            b = (_K + j) % _NBUF
            off = pl.multiple_of(g * _C, _C)
            off_nxt = pl.multiple_of((g + _K) * _C, _C)
            wait_gather(b)
            start_store(off, b)
            drain_store((b + _K) % _NBUF)
            start_gather(off_nxt, (b + _K) % _NBUF)
        return carry

    lax.fori_loop(0, _NGROUPS, group, 0)

    for g in range(_K + _NBUF * _NGROUPS, _G):
        b = g % _NBUF
        wait_gather(b)
        start_store(g * _C, b)
        drain_store((b + _K) % _NBUF)
        if g + _K < _G:
            start_gather((g + _K) * _C, (b + _K) % _NBUF)
    for g in range(_G - _K, _G):
        drain_store(g % _NBUF)


@functools.partial(jax.jit, static_argnames=())
def _gather(pos_flat, encoding):
    mesh = plsc.VectorSubcoreMesh(core_axis_name="c", subcore_axis_name="s")
    run = pl.kernel(
        _gather_body,
        out_type=jax.ShapeDtypeStruct((_B, _D), jnp.float32),
        mesh=mesh,
        scratch_types=(
            [pltpu.VMEM((_BPW,), jnp.int32)]
            + [pltpu.VMEM((_C, _D), jnp.float32) for _ in range(_NBUF)]
            + [pltpu.SemaphoreType.DMA for _ in range(2 * _NBUF)]
        ),
    )
    return run(pos_flat, encoding)


def kernel(pos, encoding):
    b, s = pos.shape
    out = _gather(pos.reshape(-1), encoding)
    return out.reshape(b, s, encoding.shape[1])
